# baseline (speedup 1.0000x reference)
import functools

import jax
import jax.numpy as jnp
from jax import lax
from jax.experimental import pallas as pl
from jax.experimental.pallas import tpu as pltpu
from jax.experimental.pallas import tpu_sc as plsc

N = 8192
D = 128
DEG = 64
K = 64
NF = 16
TEMP = 0.07
NC, NS = 2, 16
L = 16


def _sc_gather_body(z_hbm, et_hbm, nbrs_hbm, core16_hbm, nbi_hbm,
                    negi_hbm, q16_hbm,
                    zny_out, etny_out, zq_out, znx_out, znb_out, zneg_out,
                    etq_out, corenb_out, coreq_out,
                    nbi_v, negi_v, q_v, nbrows_v, ni_v, etk_v, etrows_v,
                    zrows_v, rows16f_v, nbrsq_v, crow_v, etq_v, core_v,
                    coreq_v, sem):
    wid = lax.axis_index("s") * NC + lax.axis_index("c")
    iota = lax.iota(jnp.int32, L)

    pltpu.sync_copy(nbi_hbm, nbi_v)
    pltpu.sync_copy(negi_hbm, negi_v)
    pltpu.sync_copy(q16_hbm, q_v)

    @pl.when(wid < 8)
    def _():
        kreg = plsc.load_gather(nbi_v, [8 * wid + jnp.minimum(iota, 7)])
        cp_nb = pltpu.async_copy(nbrs_hbm.at[kreg], nbrows_v, sem)
        cp_et = pltpu.async_copy(et_hbm.at[nbi_v.at[pl.ds(8 * wid, 8)]],
                                 etrows_v, sem)
        cp_nb.wait()
        cp_et.wait()
        for j in range(8):
            rsp = jnp.zeros((L,), jnp.int32) + j
            for c in range(DEG // L):
                dchunk = nbrows_v[j, pl.ds(L * c, L)]
                pos = j * DEG + L * c
                etk_v[pos // 128, pl.ds(pos % 128, L)] = plsc.load_gather(
                    etrows_v, [rsp, dchunk])
        pltpu.sync_copy(etk_v, etny_out.at[pl.ds(4 * wid, 4)])

    @pl.when((wid >= 8) & (wid < 24))
    def _():
        b = wid - 8
        kreg = plsc.load_gather(nbi_v, [4 * b + jnp.minimum(iota, 3)])
        pltpu.async_copy(nbrs_hbm.at[kreg], nbrows_v, sem).wait()
        for j in range(4):
            for c in range(DEG // L):
                pos = j * DEG + L * c
                ni_v[pos // 128, pl.ds(pos % 128, L)] = (
                    nbrows_v[j, pl.ds(L * c, L)])
        for h in range(2):
            pltpu.async_copy(z_hbm.at[ni_v.at[h]],
                             zrows_v.at[pl.ds(128 * h, 128)], sem).wait()
        pltpu.sync_copy(zrows_v, zny_out.at[pl.ds(256 * b, 256)])

    @pl.when(wid == 24)
    def _():
        qreg = q_v[...]
        pltpu.async_copy(nbrs_hbm.at[qreg], nbrows_v, sem).wait()
        for c in range(DEG // L):
            nbrsq_v[pl.ds(L * c, L)] = nbrows_v[0, pl.ds(L * c, L)]
        pltpu.async_copy(et_hbm.at[q_v.at[pl.ds(0, 1)]],
                         etrows_v.at[pl.ds(0, 1)], sem).wait()
        zsp = jnp.zeros((L,), jnp.int32)
        for c in range(DEG // L):
            sel = nbrsq_v[pl.ds(L * c, L)]
            etq_v[pl.ds(L * c, L)] = plsc.load_gather(etrows_v, [zsp, sel])
        pltpu.sync_copy(etq_v, etq_out)
        pltpu.async_copy(z_hbm.at[nbrsq_v],
                         zrows_v.at[pl.ds(0, DEG)], sem).wait()
        pltpu.sync_copy(zrows_v.at[pl.ds(0, DEG)], znx_out)
        pltpu.async_copy(z_hbm.at[qreg], zrows_v.at[pl.ds(64, L)], sem).wait()
        pltpu.sync_copy(zrows_v.at[pl.ds(64, L)], zq_out)
        for c in range(K // L):
            crow_v[pl.ds(L * c, L)] = nbi_v[pl.ds(L * c, L)] >> 4
        pltpu.async_copy(core16_hbm.at[crow_v], rows16f_v, sem).wait()
        for c in range(K // L):
            sel = nbi_v[pl.ds(L * c, L)] & (L - 1)
            core_v[pl.ds(L * c, L)] = plsc.load_gather(
                rows16f_v, [L * c + iota, sel])
        pltpu.sync_copy(core_v, corenb_out)
        pltpu.async_copy(core16_hbm.at[qreg >> 4],
                         rows16f_v.at[pl.ds(0, L)], sem).wait()
        coreq_v[...] = plsc.load_gather(rows16f_v, [iota, qreg & (L - 1)])
        pltpu.sync_copy(coreq_v, coreq_out)

    @pl.when(wid == 25)
    def _():
        pltpu.async_copy(z_hbm.at[nbi_v], zrows_v.at[pl.ds(0, K)], sem).wait()
        pltpu.sync_copy(zrows_v.at[pl.ds(0, K)], znb_out)

    @pl.when(wid == 26)
    def _():
        pltpu.async_copy(z_hbm.at[negi_v], zrows_v.at[pl.ds(0, K)],
                         sem).wait()
        pltpu.sync_copy(zrows_v.at[pl.ds(0, K)], zneg_out)


@functools.cache
def _make_sc_gather():
    mesh = plsc.VectorSubcoreMesh(
        core_axis_name="c", subcore_axis_name="s",
        num_cores=NC, num_subcores=NS)
    return functools.partial(
        pl.kernel,
        out_type=[
            jax.ShapeDtypeStruct((4096, D), jnp.float32),
            jax.ShapeDtypeStruct((32, 128), jnp.float32),
            jax.ShapeDtypeStruct((16, D), jnp.float32),
            jax.ShapeDtypeStruct((DEG, D), jnp.float32),
            jax.ShapeDtypeStruct((K, D), jnp.float32),
            jax.ShapeDtypeStruct((K, D), jnp.float32),
            jax.ShapeDtypeStruct((DEG,), jnp.float32),
            jax.ShapeDtypeStruct((K,), jnp.float32),
            jax.ShapeDtypeStruct((16,), jnp.float32),
        ],
        mesh=mesh,
        scratch_types=[
            pltpu.VMEM((K,), jnp.int32),
            pltpu.VMEM((K,), jnp.int32),
            pltpu.VMEM((L,), jnp.int32),
            pltpu.VMEM((L, DEG), jnp.int32),
            pltpu.VMEM((2, 128), jnp.int32),
            pltpu.VMEM((4, 128), jnp.float32),
            pltpu.VMEM((8, N), jnp.float32),
            pltpu.VMEM((256, D), jnp.float32),
            pltpu.VMEM((K, L), jnp.float32),
            pltpu.VMEM((DEG,), jnp.int32),
            pltpu.VMEM((K,), jnp.int32),
            pltpu.VMEM((DEG,), jnp.float32),
            pltpu.VMEM((K,), jnp.float32),
            pltpu.VMEM((L,), jnp.float32),
            pltpu.SemaphoreType.DMA,
        ],
        compiler_params=pltpu.CompilerParams(
            use_tc_tiling_on_sc=False, needs_layout_passes=False),
    )(_sc_gather_body)


def _tc_combine_body(zq16, znxr, znbr, znegr, zny, etny, etq, corenb, coreq,
                     ct, om, ph, out_ref):
    f32 = jnp.float32
    ones1 = jnp.ones((1, D), f32)
    eps = 1e-12

    zq_r = zq16[0:1, :]
    n2q = jnp.sum(zq_r * zq_r)
    invq = 1.0 / jnp.maximum(jnp.sqrt(n2q), eps)
    zq = zq_r * invq
    nsq_q = n2q * invq * invq

    def dot_t(a, b):
        return lax.dot_general(a, b, (((1,), (1,)), ((), ())),
                               preferred_element_type=f32)

    znx = znxr[...]
    znb = znbr[...]
    sqx = znx * znx
    sqb = znb * znb
    n2b_c = jnp.sum(sqb, axis=1, keepdims=True)
    invb_c = 1.0 / jnp.maximum(jnp.sqrt(n2b_c), eps)
    n2x_r = dot_t(ones1, sqx)
    invx_r = 1.0 / jnp.maximum(jnp.sqrt(n2x_r), eps)
    nsq_b_c = n2b_c * invb_c * invb_c
    nsq_x_r = n2x_r * invx_r * invx_r
    A = dot_t(znb, znx) * invb_c * invx_r
    mu_ix = 2.0 * A - nsq_b_c - nsq_x_r

    ctv = ct[0, 0]
    dtx = ctv - etq[...]
    phix = om[0, 0] * dtx + ph[0, 0]
    for j in range(1, NF):
        phix = phix + jnp.sin(om[0, j] * dtx + ph[0, j])
    w1 = jnp.exp(mu_ix * (1.0 / TEMP)) * jnp.exp(-phix)
    den1 = jnp.sum(w1, axis=1, keepdims=True) + 1e-8
    num1 = jnp.sum(w1 * mu_ix, axis=1, keepdims=True)
    term1 = num1 / den1

    s_rows = []
    n2_rows = []
    for r in range(32):
        blk = zny[128 * r:128 * r + 128, :]
        s_rows.append(dot_t(zq, blk))
        n2_rows.append(dot_t(ones1, blk * blk))
    s_raw = jnp.concatenate(s_rows, axis=0)
    n2_y = jnp.concatenate(n2_rows, axis=0)
    inv_y = 1.0 / jnp.maximum(jnp.sqrt(n2_y), eps)
    sny = s_raw * inv_y
    nsq_y = n2_y * inv_y * inv_y
    mu_y = 2.0 * sny - nsq_y - nsq_q

    dty = ctv - etny[...]
    phiy = om[0, 0] * dty + ph[0, 0]
    for j in range(1, NF):
        phiy = phiy + jnp.sin(om[0, j] * dty + ph[0, j])
    w2 = jnp.exp(mu_y * (1.0 / TEMP)) * jnp.exp(-phiy)

    ki = lax.broadcasted_iota(jnp.int32, (K, 32), 0)
    ri = lax.broadcasted_iota(jnp.int32, (K, 32), 1)
    E = jnp.where(ri == (ki >> 1), 1.0, 0.0).astype(f32)
    ki2 = lax.broadcasted_iota(jnp.int32, (K, 128), 0)
    ci2 = lax.broadcasted_iota(jnp.int32, (K, 128), 1)
    M = jnp.where((ci2 >= 64) == ((ki2 & 1) == 1), 1.0, 0.0).astype(f32)

    def seg_sum(x):
        xe = lax.dot_general(E, x, (((1,), (0,)), ((), ())),
                             preferred_element_type=f32)
        return jnp.sum(xe * M, axis=1, keepdims=True)

    den2 = seg_sum(w2) + 1e-8
    num2 = seg_sum(w2 * mu_y)
    term2 = num2 / den2

    s_nb = dot_t(znb, zq) * invb_c
    mu_xy = 2.0 * s_nb - nsq_b_c - nsq_q
    zng = znegr[...]
    n2g_c = jnp.sum(zng * zng, axis=1, keepdims=True)
    invg_c = 1.0 / jnp.maximum(jnp.sqrt(n2g_c), eps)
    nsq_g_c = n2g_c * invg_c * invg_c
    mu_neg = 2.0 * dot_t(zng, zq) * invg_c - nsq_g_c - nsq_q

    sig_p = 1.0 / (1.0 + jnp.exp(-mu_xy))
    pos_loss = -jnp.sum(jnp.log(sig_p + 1e-8)) / K
    sig_n = 1.0 / (1.0 + jnp.exp(-mu_neg))
    neg_loss = -jnp.sum(jnp.log(1.0 - sig_n + 1e-8)) / K

    cq = coreq[0, 0]
    cdiff = corenb[...] - cq
    core_loss = jnp.sum(cdiff * cdiff) / K

    delta = term1 + term2
    ad = jnp.abs(delta)
    huber = jnp.where(ad < 1.0, 0.5 * delta * delta, ad - 0.5)
    align_loss = jnp.sum(huber) / K

    total = pos_loss + neg_loss + 0.1 * core_loss + 0.1 * align_loss
    out_ref[...] = jnp.broadcast_to(total, (1, 1))


def kernel(z, query_idx, neg_idxs, neighbor_idxs, edge_times, current_time,
           neighbors, core_values, omega, phi):
    qi = jnp.asarray(query_idx, jnp.int32)
    q16 = jnp.zeros((L,), jnp.int32) + qi
    nbi = neighbor_idxs.astype(jnp.int32)
    negi = neg_idxs.astype(jnp.int32)
    nbrs = neighbors.astype(jnp.int32)

    (zny, etny, zq16, znx, znb, zneg, etq, corenb, coreq) = _make_sc_gather()(
        z, edge_times, nbrs, core_values.reshape(N // L, L), nbi, negi, q16)

    out = pl.pallas_call(
        _tc_combine_body,
        out_shape=jax.ShapeDtypeStruct((1, 1), jnp.float32),
    )(zq16, znx, znb, zneg, zny, etny,
      etq.reshape(1, DEG), corenb.reshape(1, K), coreq.reshape(1, 16),
      current_time.reshape(1, 1), omega.reshape(1, NF), phi.reshape(1, NF))
    return out.reshape(())


# speedup vs baseline: 5.7851x; 5.7851x over previous
import functools

import jax
import jax.numpy as jnp
from jax import lax
from jax.experimental import pallas as pl
from jax.experimental.pallas import tpu as pltpu
from jax.experimental.pallas import tpu_sc as plsc

N = 8192
D = 128
DEG = 64
K = 64
NF = 16
TEMP = 0.07
NC, NS = 2, 16
L = 16


def _sc_gather_body(z_hbm, et_hbm, nbrs_hbm, core128_hbm, nbi_hbm,
                    negi_hbm, q16_hbm,
                    zny_out, etny_out, zq_out, znx_out, znb_out, zneg_out,
                    etq_out, corenb_out, coreq_out,
                    nbi_v, negi_v, q_v, nbrows_v, ni_v, etk_v, etrows_v,
                    etqrow_v,
                    zrows_v, rows128f_v, nbrsq_v, crow_v, etq_v, core_v,
                    coreq_v, sem):
    wid = lax.axis_index("s") * NC + lax.axis_index("c")
    iota = lax.iota(jnp.int32, L)

    pltpu.sync_copy(nbi_hbm, nbi_v)
    pltpu.sync_copy(negi_hbm, negi_v)
    pltpu.sync_copy(q16_hbm, q_v)

    @pl.when(wid < 8)
    def _():
        kreg = plsc.load_gather(nbi_v, [8 * wid + jnp.minimum(iota, 7)])
        cp_nb = pltpu.async_copy(nbrs_hbm.at[kreg], nbrows_v, sem)
        cp_et = pltpu.async_copy(et_hbm.at[nbi_v.at[pl.ds(8 * wid, 8)]],
                                 etrows_v, sem)
        cp_nb.wait()
        cp_et.wait()
        for j in range(8):
            rsp = jnp.zeros((L,), jnp.int32) + j
            for c in range(DEG // L):
                dchunk = nbrows_v[j, pl.ds(L * c, L)]
                pos = j * DEG + L * c
                etk_v[pos // 128, pl.ds(pos % 128, L)] = plsc.load_gather(
                    etrows_v, [rsp, dchunk])
        pltpu.sync_copy(etk_v, etny_out.at[pl.ds(4 * wid, 4)])

    @pl.when((wid >= 8) & (wid < 24))
    def _():
        b = wid - 8
        kreg = plsc.load_gather(nbi_v, [4 * b + jnp.minimum(iota, 3)])
        pltpu.async_copy(nbrs_hbm.at[kreg], nbrows_v, sem).wait()
        for j in range(4):
            for c in range(DEG // L):
                pos = j * DEG + L * c
                ni_v[pos // 128, pl.ds(pos % 128, L)] = (
                    nbrows_v[j, pl.ds(L * c, L)])
        for h in range(2):
            pltpu.async_copy(z_hbm.at[ni_v.at[h]],
                             zrows_v.at[pl.ds(128 * h, 128)], sem).wait()
        pltpu.sync_copy(zrows_v, zny_out.at[pl.ds(256 * b, 256)])

    @pl.when(wid == 24)
    def _():
        qreg = q_v[...]
        pltpu.async_copy(nbrs_hbm.at[qreg], nbrows_v, sem).wait()
        for c in range(DEG // L):
            nbrsq_v[pl.ds(L * c, L)] = nbrows_v[0, pl.ds(L * c, L)]
        pltpu.async_copy(et_hbm.at[q_v.at[pl.ds(0, 1)]],
                         etqrow_v, sem).wait()
        zsp = jnp.zeros((L,), jnp.int32)
        for c in range(DEG // L):
            sel = nbrsq_v[pl.ds(L * c, L)]
            etq_v[pl.ds(L * c, L)] = plsc.load_gather(etqrow_v, [zsp, sel])
        pltpu.sync_copy(etq_v, etq_out)
        pltpu.async_copy(z_hbm.at[nbrsq_v],
                         zrows_v.at[pl.ds(0, DEG)], sem).wait()
        pltpu.sync_copy(zrows_v.at[pl.ds(0, DEG)], znx_out)
        pltpu.async_copy(z_hbm.at[qreg], zrows_v.at[pl.ds(64, L)], sem).wait()
        pltpu.sync_copy(zrows_v.at[pl.ds(64, L)], zq_out)
        for c in range(K // L):
            crow_v[pl.ds(L * c, L)] = nbi_v[pl.ds(L * c, L)] >> 7
        pltpu.async_copy(core128_hbm.at[crow_v], rows128f_v, sem).wait()
        for c in range(K // L):
            sel = nbi_v[pl.ds(L * c, L)] & 127
            core_v[pl.ds(L * c, L)] = plsc.load_gather(
                rows128f_v, [L * c + iota, sel])
        pltpu.sync_copy(core_v, corenb_out)
        pltpu.async_copy(core128_hbm.at[qreg >> 7],
                         rows128f_v.at[pl.ds(0, L)], sem).wait()
        coreq_v[...] = plsc.load_gather(rows128f_v, [iota, qreg & 127])
        pltpu.sync_copy(coreq_v, coreq_out)

    @pl.when(wid == 25)
    def _():
        pltpu.async_copy(z_hbm.at[nbi_v], zrows_v.at[pl.ds(0, K)], sem).wait()
        pltpu.sync_copy(zrows_v.at[pl.ds(0, K)], znb_out)

    @pl.when(wid == 26)
    def _():
        pltpu.async_copy(z_hbm.at[negi_v], zrows_v.at[pl.ds(0, K)],
                         sem).wait()
        pltpu.sync_copy(zrows_v.at[pl.ds(0, K)], zneg_out)


@functools.cache
def _make_sc_gather():
    mesh = plsc.VectorSubcoreMesh(
        core_axis_name="c", subcore_axis_name="s",
        num_cores=NC, num_subcores=NS)
    return functools.partial(
        pl.kernel,
        out_type=[
            jax.ShapeDtypeStruct((4096, D), jnp.float32),
            jax.ShapeDtypeStruct((32, 128), jnp.float32),
            jax.ShapeDtypeStruct((16, D), jnp.float32),
            jax.ShapeDtypeStruct((DEG, D), jnp.float32),
            jax.ShapeDtypeStruct((K, D), jnp.float32),
            jax.ShapeDtypeStruct((K, D), jnp.float32),
            jax.ShapeDtypeStruct((DEG,), jnp.float32),
            jax.ShapeDtypeStruct((K,), jnp.float32),
            jax.ShapeDtypeStruct((16,), jnp.float32),
        ],
        mesh=mesh,
        scratch_types=[
            pltpu.VMEM((K,), jnp.int32),
            pltpu.VMEM((K,), jnp.int32),
            pltpu.VMEM((L,), jnp.int32),
            pltpu.VMEM((L, 128), jnp.int32),
            pltpu.VMEM((2, 128), jnp.int32),
            pltpu.VMEM((4, 128), jnp.float32),
            pltpu.VMEM((8, N), jnp.float32),
            pltpu.VMEM((1, N), jnp.float32),
            pltpu.VMEM((256, D), jnp.float32),
            pltpu.VMEM((K, 128), jnp.float32),
            pltpu.VMEM((DEG,), jnp.int32),
            pltpu.VMEM((K,), jnp.int32),
            pltpu.VMEM((DEG,), jnp.float32),
            pltpu.VMEM((K,), jnp.float32),
            pltpu.VMEM((L,), jnp.float32),
            pltpu.SemaphoreType.DMA,
        ],
        compiler_params=pltpu.CompilerParams(needs_layout_passes=False),
    )(_sc_gather_body)


def _tc_combine_body(zq16, znxr, znbr, znegr, zny, etny, etq, corenb, coreq,
                     ct, om, ph, out_ref):
    f32 = jnp.float32
    ones1 = jnp.ones((1, D), f32)
    eps = 1e-12

    zq_r = zq16[0:1, :]
    n2q = jnp.sum(zq_r * zq_r)
    invq = 1.0 / jnp.maximum(jnp.sqrt(n2q), eps)
    zq = zq_r * invq
    nsq_q = n2q * invq * invq

    def dot_t(a, b):
        return lax.dot_general(a, b, (((1,), (1,)), ((), ())),
                               preferred_element_type=f32)

    znx = znxr[...]
    znb = znbr[...]
    sqx = znx * znx
    sqb = znb * znb
    n2b_c = jnp.sum(sqb, axis=1, keepdims=True)
    invb_c = 1.0 / jnp.maximum(jnp.sqrt(n2b_c), eps)
    n2x_r = dot_t(ones1, sqx)
    invx_r = 1.0 / jnp.maximum(jnp.sqrt(n2x_r), eps)
    nsq_b_c = n2b_c * invb_c * invb_c
    nsq_x_r = n2x_r * invx_r * invx_r
    A = dot_t(znb, znx) * invb_c * invx_r
    mu_ix = 2.0 * A - nsq_b_c - nsq_x_r

    ctv = ct[0, 0]
    dtx = ctv - etq[...]
    phix = om[0, 0] * dtx + ph[0, 0]
    for j in range(1, NF):
        phix = phix + jnp.sin(om[0, j] * dtx + ph[0, j])
    w1 = jnp.exp(mu_ix * (1.0 / TEMP)) * jnp.exp(-phix)
    den1 = jnp.sum(w1, axis=1, keepdims=True) + 1e-8
    num1 = jnp.sum(w1 * mu_ix, axis=1, keepdims=True)
    term1 = num1 / den1

    s_rows = []
    n2_rows = []
    for r in range(32):
        blk = zny[128 * r:128 * r + 128, :]
        s_rows.append(dot_t(zq, blk))
        n2_rows.append(dot_t(ones1, blk * blk))
    s_raw = jnp.concatenate(s_rows, axis=0)
    n2_y = jnp.concatenate(n2_rows, axis=0)
    inv_y = 1.0 / jnp.maximum(jnp.sqrt(n2_y), eps)
    sny = s_raw * inv_y
    nsq_y = n2_y * inv_y * inv_y
    mu_y = 2.0 * sny - nsq_y - nsq_q

    dty = ctv - etny[...]
    phiy = om[0, 0] * dty + ph[0, 0]
    for j in range(1, NF):
        phiy = phiy + jnp.sin(om[0, j] * dty + ph[0, j])
    w2 = jnp.exp(mu_y * (1.0 / TEMP)) * jnp.exp(-phiy)

    ki = lax.broadcasted_iota(jnp.int32, (K, 32), 0)
    ri = lax.broadcasted_iota(jnp.int32, (K, 32), 1)
    E = jnp.where(ri == (ki >> 1), 1.0, 0.0).astype(f32)
    ki2 = lax.broadcasted_iota(jnp.int32, (K, 128), 0)
    ci2 = lax.broadcasted_iota(jnp.int32, (K, 128), 1)
    M = jnp.where((ci2 >= 64) == ((ki2 & 1) == 1), 1.0, 0.0).astype(f32)

    def seg_sum(x):
        xe = lax.dot_general(E, x, (((1,), (0,)), ((), ())),
                             preferred_element_type=f32)
        return jnp.sum(xe * M, axis=1, keepdims=True)

    den2 = seg_sum(w2) + 1e-8
    num2 = seg_sum(w2 * mu_y)
    term2 = num2 / den2

    s_nb = dot_t(znb, zq) * invb_c
    mu_xy = 2.0 * s_nb - nsq_b_c - nsq_q
    zng = znegr[...]
    n2g_c = jnp.sum(zng * zng, axis=1, keepdims=True)
    invg_c = 1.0 / jnp.maximum(jnp.sqrt(n2g_c), eps)
    nsq_g_c = n2g_c * invg_c * invg_c
    mu_neg = 2.0 * dot_t(zng, zq) * invg_c - nsq_g_c - nsq_q

    sig_p = 1.0 / (1.0 + jnp.exp(-mu_xy))
    pos_loss = -jnp.sum(jnp.log(sig_p + 1e-8)) / K
    sig_n = 1.0 / (1.0 + jnp.exp(-mu_neg))
    neg_loss = -jnp.sum(jnp.log(1.0 - sig_n + 1e-8)) / K

    cq = coreq[0, 0]
    cdiff = corenb[...] - cq
    core_loss = jnp.sum(cdiff * cdiff) / K

    delta = term1 + term2
    ad = jnp.abs(delta)
    huber = jnp.where(ad < 1.0, 0.5 * delta * delta, ad - 0.5)
    align_loss = jnp.sum(huber) / K

    total = pos_loss + neg_loss + 0.1 * core_loss + 0.1 * align_loss
    out_ref[...] = jnp.broadcast_to(total, (1, 1))


def kernel(z, query_idx, neg_idxs, neighbor_idxs, edge_times, current_time,
           neighbors, core_values, omega, phi):
    qi = jnp.asarray(query_idx, jnp.int32)
    q16 = jnp.zeros((L,), jnp.int32) + qi
    nbi = neighbor_idxs.astype(jnp.int32)
    negi = neg_idxs.astype(jnp.int32)
    nbrs = neighbors.astype(jnp.int32)

    nbrs128 = jnp.concatenate([nbrs, jnp.zeros_like(nbrs)], axis=1)
    (zny, etny, zq16, znx, znb, zneg, etq, corenb, coreq) = _make_sc_gather()(
        z, edge_times, nbrs128, core_values.reshape(K, 128), nbi, negi, q16)

    out = pl.pallas_call(
        _tc_combine_body,
        out_shape=jax.ShapeDtypeStruct((1, 1), jnp.float32),
    )(zq16, znx, znb, zneg, zny, etny,
      etq.reshape(1, DEG), corenb.reshape(1, K), coreq.reshape(1, 16),
      current_time.reshape(1, 1), omega.reshape(1, NF), phi.reshape(1, NF))
    return out.reshape(())


# speedup vs baseline: 6.6858x; 1.1557x over previous
import functools

import jax
import jax.numpy as jnp
from jax import lax
from jax.experimental import pallas as pl
from jax.experimental.pallas import tpu as pltpu
from jax.experimental.pallas import tpu_sc as plsc

N = 8192
D = 128
DEG = 64
K = 64
NF = 16
TEMP = 0.07
NC, NS = 2, 16
L = 16


def _sc_gather_body(z_hbm, et_hbm, nbrs_hbm, core128_hbm, nbi_hbm,
                    negi_hbm, q16_hbm,
                    zny_out, etny_out, zq_out, znx_out, znb_out, zneg_out,
                    etq_out, corenb_out, coreq_out,
                    nbi_v, negi_v, q_v, nbrows_v, ni_v, etk_v, etrows_v,
                    etqrow_v, zrows_v, rows128f_v, nbrsq_v, crow_v, etq_v,
                    core_v, coreq_v, sem, sem2):
    wid = lax.axis_index("s") * NC + lax.axis_index("c")
    iota = lax.iota(jnp.int32, L)

    @pl.when(wid < 8)
    def _():
        pltpu.sync_copy(nbi_hbm, nbi_v)
        kreg = plsc.load_gather(nbi_v, [8 * wid + jnp.minimum(iota, 7)])
        cp_nb = pltpu.async_copy(nbrs_hbm.at[kreg], nbrows_v, sem)
        cp_et = pltpu.async_copy(et_hbm.at[nbi_v.at[pl.ds(8 * wid, 8)]],
                                 etrows_v, sem2)
        cp_nb.wait()
        cp_et.wait()
        for j in range(8):
            rsp = jnp.zeros((L,), jnp.int32) + j
            for c in range(DEG // L):
                dchunk = nbrows_v[j, pl.ds(L * c, L)]
                pos = j * DEG + L * c
                etk_v[pos // 128, pl.ds(pos % 128, L)] = plsc.load_gather(
                    etrows_v, [rsp, dchunk])
        pltpu.sync_copy(etk_v, etny_out.at[pl.ds(4 * wid, 4)])

    @pl.when((wid >= 8) & (wid < 24))
    def _():
        b = wid - 8
        pltpu.sync_copy(nbi_hbm, nbi_v)
        kreg = plsc.load_gather(nbi_v, [4 * b + jnp.minimum(iota, 3)])
        pltpu.async_copy(nbrs_hbm.at[kreg], nbrows_v, sem).wait()
        for j in range(4):
            for c in range(DEG // L):
                pos = j * DEG + L * c
                ni_v[pos // 128, pl.ds(pos % 128, L)] = (
                    nbrows_v[j, pl.ds(L * c, L)])
        cp0 = pltpu.async_copy(z_hbm.at[ni_v.at[0]],
                               zrows_v.at[pl.ds(0, 128)], sem)
        cp1 = pltpu.async_copy(z_hbm.at[ni_v.at[1]],
                               zrows_v.at[pl.ds(128, 128)], sem2)
        cp0.wait()
        w0 = pltpu.async_copy(zrows_v.at[pl.ds(0, 128)],
                              zny_out.at[pl.ds(256 * b, 128)], sem)
        cp1.wait()
        w1 = pltpu.async_copy(zrows_v.at[pl.ds(128, 128)],
                              zny_out.at[pl.ds(256 * b + 128, 128)], sem2)
        w0.wait()
        w1.wait()

    @pl.when(wid == 24)
    def _():
        pltpu.sync_copy(q16_hbm, q_v)
        qreg = q_v[...]
        cp_nb = pltpu.async_copy(nbrs_hbm.at[qreg], nbrows_v, sem)
        cp_et = pltpu.async_copy(et_hbm.at[q_v.at[pl.ds(0, 1)]],
                                 etqrow_v, sem2)
        cp_nb.wait()
        for c in range(DEG // L):
            nbrsq_v[pl.ds(L * c, L)] = nbrows_v[0, pl.ds(L * c, L)]
        cp_zx = pltpu.async_copy(z_hbm.at[nbrsq_v],
                                 zrows_v.at[pl.ds(0, DEG)], sem)
        cp_et.wait()
        zsp = jnp.zeros((L,), jnp.int32)
        for c in range(DEG // L):
            sel = nbrsq_v[pl.ds(L * c, L)]
            etq_v[pl.ds(L * c, L)] = plsc.load_gather(etqrow_v, [zsp, sel])
        pltpu.sync_copy(etq_v, etq_out)
        cp_zx.wait()
        pltpu.sync_copy(zrows_v.at[pl.ds(0, DEG)], znx_out)

    @pl.when(wid == 25)
    def _():
        pltpu.sync_copy(nbi_hbm, nbi_v)
        pltpu.async_copy(z_hbm.at[nbi_v], zrows_v.at[pl.ds(0, K)], sem).wait()
        pltpu.sync_copy(zrows_v.at[pl.ds(0, K)], znb_out)

    @pl.when(wid == 26)
    def _():
        pltpu.sync_copy(negi_hbm, negi_v)
        pltpu.async_copy(z_hbm.at[negi_v], zrows_v.at[pl.ds(0, K)],
                         sem).wait()
        pltpu.sync_copy(zrows_v.at[pl.ds(0, K)], zneg_out)

    @pl.when(wid == 27)
    def _():
        pltpu.sync_copy(q16_hbm, q_v)
        qreg = q_v[...]
        pltpu.async_copy(z_hbm.at[qreg], zrows_v.at[pl.ds(0, L)], sem).wait()
        pltpu.sync_copy(zrows_v.at[pl.ds(0, L)], zq_out)

    @pl.when(wid == 28)
    def _():
        pltpu.sync_copy(nbi_hbm, nbi_v)
        pltpu.sync_copy(q16_hbm, q_v)
        qreg = q_v[...]
        for c in range(K // L):
            crow_v[pl.ds(L * c, L)] = nbi_v[pl.ds(L * c, L)] >> 7
        cp_c = pltpu.async_copy(core128_hbm.at[crow_v],
                                rows128f_v.at[pl.ds(0, K)], sem)
        cp_q = pltpu.async_copy(core128_hbm.at[qreg >> 7],
                                rows128f_v.at[pl.ds(K, L)], sem2)
        cp_c.wait()
        for c in range(K // L):
            sel = nbi_v[pl.ds(L * c, L)] & 127
            core_v[pl.ds(L * c, L)] = plsc.load_gather(
                rows128f_v, [L * c + iota, sel])
        pltpu.sync_copy(core_v, corenb_out)
        cp_q.wait()
        coreq_v[...] = plsc.load_gather(rows128f_v, [K + iota, qreg & 127])
        pltpu.sync_copy(coreq_v, coreq_out)


@functools.cache
def _make_sc_gather():
    mesh = plsc.VectorSubcoreMesh(
        core_axis_name="c", subcore_axis_name="s",
        num_cores=NC, num_subcores=NS)
    return functools.partial(
        pl.kernel,
        out_type=[
            jax.ShapeDtypeStruct((4096, D), jnp.float32),
            jax.ShapeDtypeStruct((32, 128), jnp.float32),
            jax.ShapeDtypeStruct((16, D), jnp.float32),
            jax.ShapeDtypeStruct((DEG, D), jnp.float32),
            jax.ShapeDtypeStruct((K, D), jnp.float32),
            jax.ShapeDtypeStruct((K, D), jnp.float32),
            jax.ShapeDtypeStruct((DEG,), jnp.float32),
            jax.ShapeDtypeStruct((K,), jnp.float32),
            jax.ShapeDtypeStruct((16,), jnp.float32),
        ],
        mesh=mesh,
        scratch_types=[
            pltpu.VMEM((K,), jnp.int32),
            pltpu.VMEM((K,), jnp.int32),
            pltpu.VMEM((L,), jnp.int32),
            pltpu.VMEM((L, 128), jnp.int32),
            pltpu.VMEM((2, 128), jnp.int32),
            pltpu.VMEM((4, 128), jnp.float32),
            pltpu.VMEM((8, N), jnp.float32),
            pltpu.VMEM((1, N), jnp.float32),
            pltpu.VMEM((256, D), jnp.float32),
            pltpu.VMEM((K + L, 128), jnp.float32),
            pltpu.VMEM((DEG,), jnp.int32),
            pltpu.VMEM((K,), jnp.int32),
            pltpu.VMEM((DEG,), jnp.float32),
            pltpu.VMEM((K,), jnp.float32),
            pltpu.VMEM((L,), jnp.float32),
            pltpu.SemaphoreType.DMA,
            pltpu.SemaphoreType.DMA,
        ],
        compiler_params=pltpu.CompilerParams(needs_layout_passes=False),
    )(_sc_gather_body)


def _tc_combine_body(zq16, znxr, znbr, znegr, zny, etny, etq, corenb, coreq,
                     ct, om, ph, out_ref):
    f32 = jnp.float32
    ones1 = jnp.ones((1, D), f32)
    eps = 1e-12

    zq_r = zq16[0:1, :]
    n2q = jnp.sum(zq_r * zq_r)
    invq = 1.0 / jnp.maximum(jnp.sqrt(n2q), eps)
    zq = zq_r * invq
    nsq_q = n2q * invq * invq

    def dot_t(a, b):
        return lax.dot_general(a, b, (((1,), (1,)), ((), ())),
                               preferred_element_type=f32)

    znx = znxr[...]
    znb = znbr[...]
    sqx = znx * znx
    sqb = znb * znb
    n2b_c = jnp.sum(sqb, axis=1, keepdims=True)
    invb_c = 1.0 / jnp.maximum(jnp.sqrt(n2b_c), eps)
    n2x_r = dot_t(ones1, sqx)
    invx_r = 1.0 / jnp.maximum(jnp.sqrt(n2x_r), eps)
    nsq_b_c = n2b_c * invb_c * invb_c
    nsq_x_r = n2x_r * invx_r * invx_r
    A = dot_t(znb, znx) * invb_c * invx_r
    mu_ix = 2.0 * A - nsq_b_c - nsq_x_r

    ctv = ct[0, 0]
    dtx = ctv - etq[...]
    phix = om[0, 0] * dtx + ph[0, 0]
    for j in range(1, NF):
        phix = phix + jnp.sin(om[0, j] * dtx + ph[0, j])
    w1 = jnp.exp(mu_ix * (1.0 / TEMP)) * jnp.exp(-phix)
    den1 = jnp.sum(w1, axis=1, keepdims=True) + 1e-8
    num1 = jnp.sum(w1 * mu_ix, axis=1, keepdims=True)
    term1 = num1 / den1

    s_rows = []
    n2_rows = []
    for r in range(32):
        blk = zny[128 * r:128 * r + 128, :]
        s_rows.append(dot_t(zq, blk))
        n2_rows.append(dot_t(ones1, blk * blk))
    s_raw = jnp.concatenate(s_rows, axis=0)
    n2_y = jnp.concatenate(n2_rows, axis=0)
    inv_y = 1.0 / jnp.maximum(jnp.sqrt(n2_y), eps)
    sny = s_raw * inv_y
    nsq_y = n2_y * inv_y * inv_y
    mu_y = 2.0 * sny - nsq_y - nsq_q

    dty = ctv - etny[...]
    phiy = om[0, 0] * dty + ph[0, 0]
    for j in range(1, NF):
        phiy = phiy + jnp.sin(om[0, j] * dty + ph[0, j])
    w2 = jnp.exp(mu_y * (1.0 / TEMP)) * jnp.exp(-phiy)

    ki = lax.broadcasted_iota(jnp.int32, (K, 32), 0)
    ri = lax.broadcasted_iota(jnp.int32, (K, 32), 1)
    E = jnp.where(ri == (ki >> 1), 1.0, 0.0).astype(f32)
    ki2 = lax.broadcasted_iota(jnp.int32, (K, 128), 0)
    ci2 = lax.broadcasted_iota(jnp.int32, (K, 128), 1)
    M = jnp.where((ci2 >= 64) == ((ki2 & 1) == 1), 1.0, 0.0).astype(f32)

    def seg_sum(x):
        xe = lax.dot_general(E, x, (((1,), (0,)), ((), ())),
                             preferred_element_type=f32)
        return jnp.sum(xe * M, axis=1, keepdims=True)

    den2 = seg_sum(w2) + 1e-8
    num2 = seg_sum(w2 * mu_y)
    term2 = num2 / den2

    s_nb = dot_t(znb, zq) * invb_c
    mu_xy = 2.0 * s_nb - nsq_b_c - nsq_q
    zng = znegr[...]
    n2g_c = jnp.sum(zng * zng, axis=1, keepdims=True)
    invg_c = 1.0 / jnp.maximum(jnp.sqrt(n2g_c), eps)
    nsq_g_c = n2g_c * invg_c * invg_c
    mu_neg = 2.0 * dot_t(zng, zq) * invg_c - nsq_g_c - nsq_q

    sig_p = 1.0 / (1.0 + jnp.exp(-mu_xy))
    pos_loss = -jnp.sum(jnp.log(sig_p + 1e-8)) / K
    sig_n = 1.0 / (1.0 + jnp.exp(-mu_neg))
    neg_loss = -jnp.sum(jnp.log(1.0 - sig_n + 1e-8)) / K

    cq = coreq[0, 0]
    cdiff = corenb[...] - cq
    core_loss = jnp.sum(cdiff * cdiff) / K

    delta = term1 + term2
    ad = jnp.abs(delta)
    huber = jnp.where(ad < 1.0, 0.5 * delta * delta, ad - 0.5)
    align_loss = jnp.sum(huber) / K

    total = pos_loss + neg_loss + 0.1 * core_loss + 0.1 * align_loss
    out_ref[...] = jnp.broadcast_to(total, (1, 1))


def kernel(z, query_idx, neg_idxs, neighbor_idxs, edge_times, current_time,
           neighbors, core_values, omega, phi):
    qi = jnp.asarray(query_idx, jnp.int32)
    q16 = jnp.zeros((L,), jnp.int32) + qi
    nbi = neighbor_idxs.astype(jnp.int32)
    negi = neg_idxs.astype(jnp.int32)
    nbrs = neighbors.astype(jnp.int32)

    nbrs128 = jnp.concatenate([nbrs, jnp.zeros_like(nbrs)], axis=1)
    (zny, etny, zq16, znx, znb, zneg, etq, corenb, coreq) = _make_sc_gather()(
        z, edge_times, nbrs128, core_values.reshape(K, 128), nbi, negi, q16)

    out = pl.pallas_call(
        _tc_combine_body,
        out_shape=jax.ShapeDtypeStruct((1, 1), jnp.float32),
    )(zq16, znx, znb, zneg, zny, etny,
      etq.reshape(1, DEG), corenb.reshape(1, K), coreq.reshape(1, 16),
      current_time.reshape(1, 1), omega.reshape(1, NF), phi.reshape(1, NF))
    return out.reshape(())


# speedup vs baseline: 6.7359x; 1.0075x over previous
import functools

import jax
import jax.numpy as jnp
from jax import lax
from jax.experimental import pallas as pl
from jax.experimental.pallas import tpu as pltpu
from jax.experimental.pallas import tpu_sc as plsc

N = 8192
D = 128
DEG = 64
K = 64
NF = 16
TEMP = 0.07
NC, NS = 2, 16
L = 16


def _sc_gather_body(z_hbm, et_hbm, nbrs_hbm, core_hbm, nbi_hbm,
                    negi_hbm, q16_hbm,
                    zny_out, etny_out, zq_out, znx_out, znb_out, zneg_out,
                    etq_out, corenb_out, coreq_out,
                    nbi_v, negi_v, q_v, nbrows_v, ni_v, etk_v, etrows_v,
                    etqrow_v, zrows_v, core_all_v, nbrsq_v, etq_v,
                    core_v, coreq_v, sem, sem2):
    wid = lax.axis_index("s") * NC + lax.axis_index("c")
    iota = lax.iota(jnp.int32, L)

    @pl.when(wid < 8)
    def _():
        pltpu.sync_copy(nbi_hbm, nbi_v)
        kreg = plsc.load_gather(nbi_v, [8 * wid + jnp.minimum(iota, 7)])
        cp_nb = pltpu.async_copy(nbrs_hbm.at[kreg], nbrows_v, sem)
        cp_et = pltpu.async_copy(et_hbm.at[nbi_v.at[pl.ds(8 * wid, 8)]],
                                 etrows_v, sem2)
        cp_nb.wait()
        cp_et.wait()
        for j in range(8):
            rsp = jnp.zeros((L,), jnp.int32) + j
            for c in range(DEG // L):
                dchunk = nbrows_v[j, pl.ds(L * c, L)]
                pos = j * DEG + L * c
                etk_v[pos // 128, pl.ds(pos % 128, L)] = plsc.load_gather(
                    etrows_v, [rsp, dchunk])
        pltpu.sync_copy(etk_v, etny_out.at[pl.ds(4 * wid, 4)])

    @pl.when((wid >= 8) & (wid < 24))
    def _():
        b = wid - 8
        pltpu.sync_copy(nbi_hbm, nbi_v)
        kreg = plsc.load_gather(nbi_v, [4 * b + jnp.minimum(iota, 3)])
        pltpu.async_copy(nbrs_hbm.at[kreg], nbrows_v, sem).wait()
        for j in range(4):
            for c in range(DEG // L):
                pos = j * DEG + L * c
                ni_v[pos // 128, pl.ds(pos % 128, L)] = (
                    nbrows_v[j, pl.ds(L * c, L)])
        cp0 = pltpu.async_copy(z_hbm.at[ni_v.at[0]],
                               zrows_v.at[pl.ds(0, 128)], sem)
        cp1 = pltpu.async_copy(z_hbm.at[ni_v.at[1]],
                               zrows_v.at[pl.ds(128, 128)], sem2)
        cp0.wait()
        w0 = pltpu.async_copy(zrows_v.at[pl.ds(0, 128)],
                              zny_out.at[pl.ds(256 * b, 128)], sem)
        cp1.wait()
        w1 = pltpu.async_copy(zrows_v.at[pl.ds(128, 128)],
                              zny_out.at[pl.ds(256 * b + 128, 128)], sem2)
        w0.wait()
        w1.wait()

    @pl.when(wid == 24)
    def _():
        pltpu.sync_copy(q16_hbm, q_v)
        qreg = q_v[...]
        cp_nb = pltpu.async_copy(nbrs_hbm.at[qreg], nbrows_v, sem)
        cp_et = pltpu.async_copy(et_hbm.at[q_v.at[pl.ds(0, 1)]],
                                 etqrow_v, sem2)
        cp_nb.wait()
        for c in range(DEG // L):
            nbrsq_v[pl.ds(L * c, L)] = nbrows_v[0, pl.ds(L * c, L)]
        cp_zx = pltpu.async_copy(z_hbm.at[nbrsq_v],
                                 zrows_v.at[pl.ds(0, DEG)], sem)
        cp_et.wait()
        zsp = jnp.zeros((L,), jnp.int32)
        for c in range(DEG // L):
            sel = nbrsq_v[pl.ds(L * c, L)]
            etq_v[pl.ds(L * c, L)] = plsc.load_gather(etqrow_v, [zsp, sel])
        pltpu.sync_copy(etq_v, etq_out)
        cp_zx.wait()
        pltpu.sync_copy(zrows_v.at[pl.ds(0, DEG)], znx_out)

    @pl.when(wid == 25)
    def _():
        pltpu.sync_copy(nbi_hbm, nbi_v)
        pltpu.async_copy(z_hbm.at[nbi_v], zrows_v.at[pl.ds(0, K)], sem).wait()
        pltpu.sync_copy(zrows_v.at[pl.ds(0, K)], znb_out)

    @pl.when(wid == 26)
    def _():
        pltpu.sync_copy(negi_hbm, negi_v)
        pltpu.async_copy(z_hbm.at[negi_v], zrows_v.at[pl.ds(0, K)],
                         sem).wait()
        pltpu.sync_copy(zrows_v.at[pl.ds(0, K)], zneg_out)

    @pl.when(wid == 27)
    def _():
        pltpu.sync_copy(q16_hbm, q_v)
        qreg = q_v[...]
        pltpu.async_copy(z_hbm.at[qreg], zrows_v.at[pl.ds(0, L)], sem).wait()
        pltpu.sync_copy(zrows_v.at[pl.ds(0, L)], zq_out)

    @pl.when(wid == 28)
    def _():
        pltpu.sync_copy(nbi_hbm, nbi_v)
        pltpu.sync_copy(q16_hbm, q_v)
        qreg = q_v[...]
        pltpu.sync_copy(core_hbm, core_all_v)
        for c in range(K // L):
            core_v[pl.ds(L * c, L)] = plsc.load_gather(
                core_all_v, [nbi_v[pl.ds(L * c, L)]])
        pltpu.sync_copy(core_v, corenb_out)
        coreq_v[...] = plsc.load_gather(core_all_v, [qreg])
        pltpu.sync_copy(coreq_v, coreq_out)


@functools.cache
def _make_sc_gather():
    mesh = plsc.VectorSubcoreMesh(
        core_axis_name="c", subcore_axis_name="s",
        num_cores=NC, num_subcores=NS)
    return functools.partial(
        pl.kernel,
        out_type=[
            jax.ShapeDtypeStruct((4096, D), jnp.float32),
            jax.ShapeDtypeStruct((32, 128), jnp.float32),
            jax.ShapeDtypeStruct((16, D), jnp.float32),
            jax.ShapeDtypeStruct((DEG, D), jnp.float32),
            jax.ShapeDtypeStruct((K, D), jnp.float32),
            jax.ShapeDtypeStruct((K, D), jnp.float32),
            jax.ShapeDtypeStruct((DEG,), jnp.float32),
            jax.ShapeDtypeStruct((K,), jnp.float32),
            jax.ShapeDtypeStruct((16,), jnp.float32),
        ],
        mesh=mesh,
        scratch_types=[
            pltpu.VMEM((K,), jnp.int32),
            pltpu.VMEM((K,), jnp.int32),
            pltpu.VMEM((L,), jnp.int32),
            pltpu.VMEM((L, 128), jnp.int32),
            pltpu.VMEM((2, 128), jnp.int32),
            pltpu.VMEM((4, 128), jnp.float32),
            pltpu.VMEM((8, N), jnp.float32),
            pltpu.VMEM((1, N), jnp.float32),
            pltpu.VMEM((256, D), jnp.float32),
            pltpu.VMEM((N,), jnp.float32),
            pltpu.VMEM((DEG,), jnp.int32),
            pltpu.VMEM((DEG,), jnp.float32),
            pltpu.VMEM((K,), jnp.float32),
            pltpu.VMEM((L,), jnp.float32),
            pltpu.SemaphoreType.DMA,
            pltpu.SemaphoreType.DMA,
        ],
        compiler_params=pltpu.CompilerParams(needs_layout_passes=False),
    )(_sc_gather_body)


def _tc_combine_body(zq16, znxr, znbr, znegr, zny, etny, etq, corenb, coreq,
                     ct, om, ph, out_ref):
    f32 = jnp.float32
    ones1 = jnp.ones((1, D), f32)
    eps = 1e-12

    zq_r = zq16[0:1, :]
    n2q = jnp.sum(zq_r * zq_r)
    invq = 1.0 / jnp.maximum(jnp.sqrt(n2q), eps)
    zq = zq_r * invq
    nsq_q = n2q * invq * invq

    def dot_t(a, b):
        return lax.dot_general(a, b, (((1,), (1,)), ((), ())),
                               preferred_element_type=f32)

    znx = znxr[...]
    znb = znbr[...]
    sqx = znx * znx
    sqb = znb * znb
    n2b_c = jnp.sum(sqb, axis=1, keepdims=True)
    invb_c = 1.0 / jnp.maximum(jnp.sqrt(n2b_c), eps)
    n2x_r = dot_t(ones1, sqx)
    invx_r = 1.0 / jnp.maximum(jnp.sqrt(n2x_r), eps)
    nsq_b_c = n2b_c * invb_c * invb_c
    nsq_x_r = n2x_r * invx_r * invx_r
    A = dot_t(znb, znx) * invb_c * invx_r
    mu_ix = 2.0 * A - nsq_b_c - nsq_x_r

    ctv = ct[0, 0]
    dtx = ctv - etq[...]
    phix = om[0, 0] * dtx + ph[0, 0]
    for j in range(1, NF):
        phix = phix + jnp.sin(om[0, j] * dtx + ph[0, j])
    w1 = jnp.exp(mu_ix * (1.0 / TEMP)) * jnp.exp(-phix)
    den1 = jnp.sum(w1, axis=1, keepdims=True) + 1e-8
    num1 = jnp.sum(w1 * mu_ix, axis=1, keepdims=True)
    term1 = num1 / den1

    s_rows = []
    n2_rows = []
    for r in range(32):
        blk = zny[128 * r:128 * r + 128, :]
        s_rows.append(dot_t(zq, blk))
        n2_rows.append(dot_t(ones1, blk * blk))
    s_raw = jnp.concatenate(s_rows, axis=0)
    n2_y = jnp.concatenate(n2_rows, axis=0)
    inv_y = 1.0 / jnp.maximum(jnp.sqrt(n2_y), eps)
    sny = s_raw * inv_y
    nsq_y = n2_y * inv_y * inv_y
    mu_y = 2.0 * sny - nsq_y - nsq_q

    dty = ctv - etny[...]
    phiy = om[0, 0] * dty + ph[0, 0]
    for j in range(1, NF):
        phiy = phiy + jnp.sin(om[0, j] * dty + ph[0, j])
    w2 = jnp.exp(mu_y * (1.0 / TEMP)) * jnp.exp(-phiy)

    ki = lax.broadcasted_iota(jnp.int32, (K, 32), 0)
    ri = lax.broadcasted_iota(jnp.int32, (K, 32), 1)
    E = jnp.where(ri == (ki >> 1), 1.0, 0.0).astype(f32)
    ki2 = lax.broadcasted_iota(jnp.int32, (K, 128), 0)
    ci2 = lax.broadcasted_iota(jnp.int32, (K, 128), 1)
    M = jnp.where((ci2 >= 64) == ((ki2 & 1) == 1), 1.0, 0.0).astype(f32)

    def seg_sum(x):
        xe = lax.dot_general(E, x, (((1,), (0,)), ((), ())),
                             preferred_element_type=f32)
        return jnp.sum(xe * M, axis=1, keepdims=True)

    den2 = seg_sum(w2) + 1e-8
    num2 = seg_sum(w2 * mu_y)
    term2 = num2 / den2

    s_nb = dot_t(znb, zq) * invb_c
    mu_xy = 2.0 * s_nb - nsq_b_c - nsq_q
    zng = znegr[...]
    n2g_c = jnp.sum(zng * zng, axis=1, keepdims=True)
    invg_c = 1.0 / jnp.maximum(jnp.sqrt(n2g_c), eps)
    nsq_g_c = n2g_c * invg_c * invg_c
    mu_neg = 2.0 * dot_t(zng, zq) * invg_c - nsq_g_c - nsq_q

    sig_p = 1.0 / (1.0 + jnp.exp(-mu_xy))
    pos_loss = -jnp.sum(jnp.log(sig_p + 1e-8)) / K
    sig_n = 1.0 / (1.0 + jnp.exp(-mu_neg))
    neg_loss = -jnp.sum(jnp.log(1.0 - sig_n + 1e-8)) / K

    cq = coreq[0, 0]
    cdiff = corenb[...] - cq
    core_loss = jnp.sum(cdiff * cdiff) / K

    delta = term1 + term2
    ad = jnp.abs(delta)
    huber = jnp.where(ad < 1.0, 0.5 * delta * delta, ad - 0.5)
    align_loss = jnp.sum(huber) / K

    total = pos_loss + neg_loss + 0.1 * core_loss + 0.1 * align_loss
    out_ref[...] = jnp.broadcast_to(total, (1, 1))


def kernel(z, query_idx, neg_idxs, neighbor_idxs, edge_times, current_time,
           neighbors, core_values, omega, phi):
    qi = jnp.asarray(query_idx, jnp.int32)
    q16 = jnp.zeros((L,), jnp.int32) + qi
    nbi = neighbor_idxs.astype(jnp.int32)
    negi = neg_idxs.astype(jnp.int32)
    nbrs = neighbors.astype(jnp.int32)

    nbrs128 = jnp.concatenate([nbrs, jnp.zeros_like(nbrs)], axis=1)
    (zny, etny, zq16, znx, znb, zneg, etq, corenb, coreq) = _make_sc_gather()(
        z, edge_times, nbrs128, core_values, nbi, negi, q16)

    out = pl.pallas_call(
        _tc_combine_body,
        out_shape=jax.ShapeDtypeStruct((1, 1), jnp.float32),
    )(zq16, znx, znb, zneg, zny, etny,
      etq.reshape(1, DEG), corenb.reshape(1, K), coreq.reshape(1, 16),
      current_time.reshape(1, 1), omega.reshape(1, NF), phi.reshape(1, NF))
    return out.reshape(())
